# Initial kernel scaffold; baseline (speedup 1.0000x reference)
import jax
import jax.numpy as jnp
from jax.experimental import pallas as pl


def kernel(x, W1, b1, W2, b2, training):
    raise NotImplementedError("write your pallas kernel here")


# speedup vs baseline: 1.5067x; 1.5067x over previous
import functools

import jax
import jax.numpy as jnp
from jax import lax
from jax.experimental import pallas as pl
from jax.experimental.pallas import tpu as pltpu

INPUT_DIM = 768
HIDDEN_DIM = 384
NUM_EXPERTS = 64
TOP_K = 8
_INV_SQRT2 = 0.7071067811865476


def _gate_body(x_ref, w1_ref, b1_ref, w2_ref, b2_ref,
               gates_ref, idx_ref, loss_ref, acc_ref, *, n_tokens):
    i = pl.program_id(0)
    nsteps = pl.num_programs(0)

    x = x_ref[...]
    h = jnp.dot(x, w1_ref[...], preferred_element_type=jnp.float32)
    h = h + b1_ref[...]
    h = 0.5 * h * (1.0 + lax.erf(h * _INV_SQRT2))
    logits = jnp.dot(h, w2_ref[...], preferred_element_type=jnp.float32)
    logits = logits + b2_ref[...]

    t = logits.shape[0]
    e = logits.shape[1]
    col = lax.broadcasted_iota(jnp.int32, (t, e), 1)

    m_all = jnp.max(logits, axis=1, keepdims=True)
    p = jnp.exp(logits - m_all)
    probs = p / jnp.sum(p, axis=1, keepdims=True)
    colsum = jnp.sum(probs, axis=0, keepdims=True)

    @pl.when(i == 0)
    def _():
        acc_ref[...] = jnp.zeros_like(acc_ref)

    acc_ref[...] += colsum

    work = logits
    vals = []
    idxs = []
    for _ in range(TOP_K):
        mj = jnp.max(work, axis=1, keepdims=True)
        ij = jnp.min(jnp.where(work == mj, col, e), axis=1, keepdims=True)
        vals.append(mj)
        idxs.append(ij)
        work = jnp.where(col == ij, -jnp.inf, work)

    topv = jnp.concatenate(vals, axis=1)
    topi = jnp.concatenate(idxs, axis=1).astype(jnp.int32)

    g = jnp.exp(topv - topv[:, :1])
    gates_ref[...] = g / jnp.sum(g, axis=1, keepdims=True)
    idx_ref[...] = topi

    @pl.when(i == nsteps - 1)
    def _():
        mean_probs = acc_ref[...] * (1.0 / n_tokens)
        diff = mean_probs - (1.0 / NUM_EXPERTS)
        loss_ref[...] = jnp.sum(diff * diff, keepdims=True).reshape(1, 1) * (
            1.0 / NUM_EXPERTS)


def kernel(x, W1, b1, W2, b2, training=0):
    n = x.shape[0] * x.shape[1]
    x_flat = x.reshape(n, x.shape[2])
    block_t = 1024
    grid = (n // block_t,)

    gates, idx, loss = pl.pallas_call(
        functools.partial(_gate_body, n_tokens=n),
        grid=grid,
        in_specs=[
            pl.BlockSpec((block_t, INPUT_DIM), lambda i: (i, 0)),
            pl.BlockSpec((INPUT_DIM, HIDDEN_DIM), lambda i: (0, 0)),
            pl.BlockSpec((1, HIDDEN_DIM), lambda i: (0, 0)),
            pl.BlockSpec((HIDDEN_DIM, NUM_EXPERTS), lambda i: (0, 0)),
            pl.BlockSpec((1, NUM_EXPERTS), lambda i: (0, 0)),
        ],
        out_specs=[
            pl.BlockSpec((block_t, TOP_K), lambda i: (i, 0)),
            pl.BlockSpec((block_t, TOP_K), lambda i: (i, 0)),
            pl.BlockSpec((1, 1), lambda i: (0, 0)),
        ],
        out_shape=[
            jax.ShapeDtypeStruct((n, TOP_K), jnp.float32),
            jax.ShapeDtypeStruct((n, TOP_K), jnp.int32),
            jax.ShapeDtypeStruct((1, 1), jnp.float32),
        ],
        scratch_shapes=[pltpu.VMEM((1, NUM_EXPERTS), jnp.float32)],
    )(x_flat, W1, b1.reshape(1, HIDDEN_DIM), W2, b2.reshape(1, NUM_EXPERTS))

    return gates, idx, loss[0, 0]


# speedup vs baseline: 3.0235x; 2.0067x over previous
import functools

import jax
import jax.numpy as jnp
from jax import lax
from jax.experimental import pallas as pl
from jax.experimental.pallas import tpu as pltpu

INPUT_DIM = 768
HIDDEN_DIM = 384
NUM_EXPERTS = 64
TOP_K = 8
_INV_SQRT2 = 0.7071067811865476


def _gate_body(x_ref, w1_ref, b1_ref, w2t_ref, b2_ref,
               gates_ref, idx_ref, loss_ref, acc_ref, *, n_tokens):
    i = pl.program_id(0)
    nsteps = pl.num_programs(0)

    x = x_ref[...]
    h = jnp.dot(x, w1_ref[...], preferred_element_type=jnp.float32)
    h = h + b1_ref[...]
    h = 0.5 * h * (1.0 + lax.erf(h * _INV_SQRT2))
    logits_t = lax.dot_general(
        w2t_ref[...], h,
        dimension_numbers=(((1,), (1,)), ((), ())),
        preferred_element_type=jnp.float32,
    )
    logits_t = logits_t + b2_ref[...]

    e = logits_t.shape[0]
    t = logits_t.shape[1]
    row_f = lax.broadcasted_iota(jnp.int32, (e, t), 0).astype(jnp.float32)

    work = logits_t
    vals = []
    idxs = []
    for _ in range(TOP_K):
        mj = jnp.max(work, axis=0, keepdims=True)
        ij = jnp.min(jnp.where(work == mj, row_f, 128.0),
                     axis=0, keepdims=True)
        vals.append(mj)
        idxs.append(ij)
        work = jnp.where(row_f == ij, -jnp.inf, work)

    topv = jnp.concatenate(vals, axis=0)
    topi = jnp.concatenate(idxs, axis=0).astype(jnp.int32)

    g = jnp.exp(topv - topv[0:1, :])
    g = g * (1.0 / jnp.sum(g, axis=0, keepdims=True))
    gates_ref[...] = g.T
    idx_ref[...] = topi.T

    p = jnp.exp(logits_t - vals[0])
    probs = p * (1.0 / jnp.sum(p, axis=0, keepdims=True))

    lanes = acc_ref.shape[1]
    psum = probs[:, 0:lanes]
    for c in range(1, t // lanes):
        psum = psum + probs[:, c * lanes:(c + 1) * lanes]

    @pl.when(i == 0)
    def _():
        acc_ref[...] = jnp.zeros_like(acc_ref)

    acc_ref[...] += psum

    @pl.when(i == nsteps - 1)
    def _():
        mean_probs = jnp.sum(acc_ref[...], axis=1, keepdims=True) * (
            1.0 / n_tokens)
        diff = mean_probs - (1.0 / NUM_EXPERTS)
        loss_ref[...] = jnp.sum(diff * diff, keepdims=True).reshape(1, 1) * (
            1.0 / NUM_EXPERTS)


def kernel(x, W1, b1, W2, b2, training=0):
    n = x.shape[0] * x.shape[1]
    x_flat = x.reshape(n, x.shape[2])
    block_t = 1024
    grid = (n // block_t,)

    gates, idx, loss = pl.pallas_call(
        functools.partial(_gate_body, n_tokens=n),
        grid=grid,
        in_specs=[
            pl.BlockSpec((block_t, INPUT_DIM), lambda i: (i, 0)),
            pl.BlockSpec((INPUT_DIM, HIDDEN_DIM), lambda i: (0, 0)),
            pl.BlockSpec((1, HIDDEN_DIM), lambda i: (0, 0)),
            pl.BlockSpec((NUM_EXPERTS, HIDDEN_DIM), lambda i: (0, 0)),
            pl.BlockSpec((NUM_EXPERTS, 1), lambda i: (0, 0)),
        ],
        out_specs=[
            pl.BlockSpec((block_t, TOP_K), lambda i: (i, 0)),
            pl.BlockSpec((block_t, TOP_K), lambda i: (i, 0)),
            pl.BlockSpec((1, 1), lambda i: (0, 0)),
        ],
        out_shape=[
            jax.ShapeDtypeStruct((n, TOP_K), jnp.float32),
            jax.ShapeDtypeStruct((n, TOP_K), jnp.int32),
            jax.ShapeDtypeStruct((1, 1), jnp.float32),
        ],
        scratch_shapes=[pltpu.VMEM((NUM_EXPERTS, 128), jnp.float32)],
    )(x_flat, W1, b1.reshape(1, HIDDEN_DIM), W2.T,
      b2.reshape(NUM_EXPERTS, 1))

    return gates, idx, loss[0, 0]


# speedup vs baseline: 3.3036x; 1.0926x over previous
import functools

import jax
import jax.numpy as jnp
from jax import lax
from jax.experimental import pallas as pl
from jax.experimental.pallas import tpu as pltpu

INPUT_DIM = 768
HIDDEN_DIM = 384
NUM_EXPERTS = 64
TOP_K = 8
_INV_SQRT2 = 0.7071067811865476


def _gate_body(x_ref, w1_ref, b1_ref, w2t_ref, b2_ref,
               gates_ref, idx_ref, loss_ref, acc_ref, *, n_tokens):
    i = pl.program_id(0)
    nsteps = pl.num_programs(0)

    x = x_ref[...]
    h = jnp.dot(x, w1_ref[...], preferred_element_type=jnp.float32)
    h = h + b1_ref[...]
    h = 0.5 * h * (1.0 + lax.erf(h * _INV_SQRT2))
    logits_t = lax.dot_general(
        w2t_ref[...], h,
        dimension_numbers=(((1,), (1,)), ((), ())),
        preferred_element_type=jnp.float32,
    )
    logits_t = logits_t + b2_ref[...]

    e = logits_t.shape[0]
    t = logits_t.shape[1]
    row_f = lax.broadcasted_iota(jnp.int32, (e, t), 0).astype(jnp.float32)

    work = logits_t
    vals = []
    idxs = []
    for _ in range(TOP_K):
        mj = jnp.max(work, axis=0, keepdims=True)
        ij = jnp.min(jnp.where(work == mj, row_f, 128.0),
                     axis=0, keepdims=True)
        vals.append(mj)
        idxs.append(ij)
        work = jnp.where(row_f == ij, -jnp.inf, work)

    topv = jnp.concatenate(vals, axis=0)
    topi = jnp.concatenate(idxs, axis=0).astype(jnp.int32)

    g = jnp.exp(topv - topv[0:1, :])
    g = g * (1.0 / jnp.sum(g, axis=0, keepdims=True))
    gates_ref[...] = g.T
    idx_ref[...] = topi.T

    p = jnp.exp(logits_t - vals[0])
    probs = p * (1.0 / jnp.sum(p, axis=0, keepdims=True))

    lanes = acc_ref.shape[1]
    psum = probs[:, 0:lanes]
    for c in range(1, t // lanes):
        psum = psum + probs[:, c * lanes:(c + 1) * lanes]

    @pl.when(i == 0)
    def _():
        acc_ref[...] = jnp.zeros_like(acc_ref)

    acc_ref[...] += psum

    @pl.when(i == nsteps - 1)
    def _():
        mean_probs = jnp.sum(acc_ref[...], axis=1, keepdims=True) * (
            1.0 / n_tokens)
        diff = mean_probs - (1.0 / NUM_EXPERTS)
        loss_ref[...] = jnp.sum(diff * diff, keepdims=True).reshape(1, 1) * (
            1.0 / NUM_EXPERTS)


def kernel(x, W1, b1, W2, b2, training=0):
    n = x.shape[0] * x.shape[1]
    x_flat = x.reshape(n, x.shape[2])
    block_t = 2048
    grid = (n // block_t,)

    gates, idx, loss = pl.pallas_call(
        functools.partial(_gate_body, n_tokens=n),
        grid=grid,
        in_specs=[
            pl.BlockSpec((block_t, INPUT_DIM), lambda i: (i, 0)),
            pl.BlockSpec((INPUT_DIM, HIDDEN_DIM), lambda i: (0, 0)),
            pl.BlockSpec((1, HIDDEN_DIM), lambda i: (0, 0)),
            pl.BlockSpec((NUM_EXPERTS, HIDDEN_DIM), lambda i: (0, 0)),
            pl.BlockSpec((NUM_EXPERTS, 1), lambda i: (0, 0)),
        ],
        out_specs=[
            pl.BlockSpec((block_t, TOP_K), lambda i: (i, 0)),
            pl.BlockSpec((block_t, TOP_K), lambda i: (i, 0)),
            pl.BlockSpec((1, 1), lambda i: (0, 0)),
        ],
        out_shape=[
            jax.ShapeDtypeStruct((n, TOP_K), jnp.float32),
            jax.ShapeDtypeStruct((n, TOP_K), jnp.int32),
            jax.ShapeDtypeStruct((1, 1), jnp.float32),
        ],
        scratch_shapes=[pltpu.VMEM((NUM_EXPERTS, 128), jnp.float32)],
    )(x_flat, W1, b1.reshape(1, HIDDEN_DIM), W2.T,
      b2.reshape(NUM_EXPERTS, 1))

    return gates, idx, loss[0, 0]


# speedup vs baseline: 3.3525x; 1.0148x over previous
import functools

import jax
import jax.numpy as jnp
from jax import lax
from jax.experimental import pallas as pl
from jax.experimental.pallas import tpu as pltpu

INPUT_DIM = 768
HIDDEN_DIM = 384
NUM_EXPERTS = 64
TOP_K = 8
_INV_SQRT2 = 0.7071067811865476


def _gate_body(x_ref, w1_ref, b1_ref, w2t_ref, b2_ref,
               gates_ref, idx_ref, loss_ref, acc_ref, *, n_tokens):
    i = pl.program_id(0)
    nsteps = pl.num_programs(0)

    x = x_ref[...]
    h = jnp.dot(x, w1_ref[...], preferred_element_type=jnp.float32)
    h = h + b1_ref[...]
    h = 0.5 * h * (1.0 + lax.erf(h * _INV_SQRT2))
    logits_t = lax.dot_general(
        w2t_ref[...], h,
        dimension_numbers=(((1,), (1,)), ((), ())),
        preferred_element_type=jnp.float32,
    )
    logits_t = logits_t + b2_ref[...]

    e = logits_t.shape[0]
    t = logits_t.shape[1]
    row_f = lax.broadcasted_iota(jnp.int32, (e, t), 0).astype(jnp.float32)

    work = logits_t
    vals = []
    idxs = []
    for _ in range(TOP_K):
        mj = jnp.max(work, axis=0, keepdims=True)
        ij = jnp.min(jnp.where(work == mj, row_f, 128.0),
                     axis=0, keepdims=True)
        vals.append(mj)
        idxs.append(ij)
        work = jnp.where(row_f == ij, -jnp.inf, work)

    topv = jnp.concatenate(vals, axis=0)
    topi = jnp.concatenate(idxs, axis=0).astype(jnp.int32)

    g = jnp.exp(topv - topv[0:1, :])
    g = g * (1.0 / jnp.sum(g, axis=0, keepdims=True))
    gates_ref[...] = g.T
    idx_ref[...] = topi.T

    p = jnp.exp(logits_t - vals[0])
    probs = p * (1.0 / jnp.sum(p, axis=0, keepdims=True))

    lanes = acc_ref.shape[1]
    psum = probs[:, 0:lanes]
    for c in range(1, t // lanes):
        psum = psum + probs[:, c * lanes:(c + 1) * lanes]

    @pl.when(i == 0)
    def _():
        acc_ref[...] = jnp.zeros_like(acc_ref)

    acc_ref[...] += psum

    @pl.when(i == nsteps - 1)
    def _():
        mean_probs = jnp.sum(acc_ref[...], axis=1, keepdims=True) * (
            1.0 / n_tokens)
        diff = mean_probs - (1.0 / NUM_EXPERTS)
        loss_ref[...] = jnp.sum(diff * diff, keepdims=True).reshape(1, 1) * (
            1.0 / NUM_EXPERTS)


def kernel(x, W1, b1, W2, b2, training=0):
    n = x.shape[0] * x.shape[1]
    x_flat = x.reshape(n, x.shape[2])
    block_t = 4096
    grid = (n // block_t,)

    gates, idx, loss = pl.pallas_call(
        functools.partial(_gate_body, n_tokens=n),
        grid=grid,
        in_specs=[
            pl.BlockSpec((block_t, INPUT_DIM), lambda i: (i, 0)),
            pl.BlockSpec((INPUT_DIM, HIDDEN_DIM), lambda i: (0, 0)),
            pl.BlockSpec((1, HIDDEN_DIM), lambda i: (0, 0)),
            pl.BlockSpec((NUM_EXPERTS, HIDDEN_DIM), lambda i: (0, 0)),
            pl.BlockSpec((NUM_EXPERTS, 1), lambda i: (0, 0)),
        ],
        out_specs=[
            pl.BlockSpec((block_t, TOP_K), lambda i: (i, 0)),
            pl.BlockSpec((block_t, TOP_K), lambda i: (i, 0)),
            pl.BlockSpec((1, 1), lambda i: (0, 0)),
        ],
        out_shape=[
            jax.ShapeDtypeStruct((n, TOP_K), jnp.float32),
            jax.ShapeDtypeStruct((n, TOP_K), jnp.int32),
            jax.ShapeDtypeStruct((1, 1), jnp.float32),
        ],
        scratch_shapes=[pltpu.VMEM((NUM_EXPERTS, 128), jnp.float32)],
    )(x_flat, W1, b1.reshape(1, HIDDEN_DIM), W2.T,
      b2.reshape(NUM_EXPERTS, 1))

    return gates, idx, loss[0, 0]
